# baseline (speedup 1.0000x reference)
import functools

import jax
import jax.numpy as jnp
from jax import lax
from jax.experimental import pallas as pl
from jax.experimental.pallas import tpu as pltpu
from jax.experimental.pallas import tpu_sc as plsc

_R = 200


def _make_gather(B, L, V, D, num_cores, num_subcores):
    NW = num_cores * num_subcores
    b_per_w = B // NW
    n_blocks = V // _R
    blocks_per_w = -(-n_blocks // num_subcores)
    nq = D // 16
    mesh = plsc.VectorSubcoreMesh(core_axis_name="c", subcore_axis_name="s")

    @functools.partial(
        pl.kernel,
        mesh=mesh,
        out_type=jax.ShapeDtypeStruct((B, L, D), jnp.float32),
        scratch_types=[
            pltpu.HBM((V, 2 * D), jnp.float32),
            pltpu.VMEM((b_per_w * L,), jnp.int32),
            pltpu.VMEM((8, L), jnp.int32),
            pltpu.VMEM((2, L, 2 * D), jnp.float32),
            pltpu.VMEM((2, _R, D), jnp.float32),
            pltpu.VMEM((16,), jnp.int32),
            pltpu.VMEM((16,), jnp.int32),
            pltpu.SemaphoreType.DMA((2,)),
            pltpu.SemaphoreType.DMA((2,)),
            pltpu.SemaphoreType.DMA,
        ],
    )
    def gather_kernel(idx_hbm, table_hbm, out_hbm,
                      scratch, idx_v, piece_v, wide_v, slim_v, flag_v, poll_v,
                      gsem, ssem, isem):
        sc = lax.axis_index("c")
        sid = lax.axis_index("s")
        wid = sid * num_cores + sc
        brow0 = wid * b_per_w

        qoffs = [q * 16 for q in range(L // 16)] + [L - 16]

        def _ipiece(pc, carry):
            pltpu.sync_copy(idx_hbm.at[pl.ds(brow0 + pc * 8, 8), :], piece_v)

            def _irow(r2, c2):
                for q0 in qoffs:
                    idx_v[pl.ds((pc * 8 + r2) * L + q0, 16)] = (
                        piece_v[r2, pl.ds(q0, 16)])
                return c2

            lax.fori_loop(0, 8, _irow, 0)
            return carry

        lax.fori_loop(0, b_per_w // 8, _ipiece, 0)

        def blk(k):
            return jnp.minimum(sid + k * num_subcores, n_blocks - 1) * _R

        def read_copy(k, p):
            return pltpu.make_async_copy(
                table_hbm.at[pl.ds(blk(k), _R), :], slim_v.at[p], gsem.at[p])

        def swrite_copy(k, p):
            return pltpu.make_async_copy(
                wide_v.at[p, pl.ds(0, _R), :],
                scratch.at[pl.ds(blk(k), _R), :], ssem.at[p])

        def widen(p, nrows):
            def rbody(r, carry):
                for q in range(nq):
                    wide_v[p, r, pl.ds(q * 16, 16)] = (
                        slim_v[p, r, pl.ds(q * 16, 16)])
                return carry

            lax.fori_loop(0, nrows, rbody, 0)

        read_copy(0, 0).start()

        def dbody(k, carry):
            p = lax.rem(k, 2)
            read_copy(k, p).wait()

            @pl.when(k + 1 < blocks_per_w)
            def _next_read():
                read_copy(k + 1, 1 - p).start()

            @pl.when(k >= 2)
            def _recycle():
                swrite_copy(k - 2, p).wait()

            widen(p, _R)
            swrite_copy(k, p).start()
            return carry

        lax.fori_loop(0, blocks_per_w, dbody, 0)
        swrite_copy(blocks_per_w - 2, lax.rem(blocks_per_w - 2, 2)).wait()
        swrite_copy(blocks_per_w - 1, lax.rem(blocks_per_w - 1, 2)).wait()

        plsc.subcore_barrier()

        def gather_copy(i, b):
            return pltpu.make_async_copy(
                scratch.at[idx_v.at[pl.ds(i * L, L)]], wide_v.at[b],
                gsem.at[b])

        def store_copy(i, b):
            return pltpu.make_async_copy(
                slim_v.at[b], out_hbm.at[brow0 + i], ssem.at[b])

        def slim(b):
            def rbody(r, carry):
                for q in range(nq):
                    slim_v[b, r, pl.ds(q * 16, 16)] = (
                        wide_v[b, r, pl.ds(q * 16, 16)])
                return carry

            lax.fori_loop(0, L, rbody, 0)

        gather_copy(0, 0).start()
        gather_copy(1, 1).start()

        def body(i, carry):
            b = lax.rem(i, 2)
            gather_copy(i, b).wait()

            @pl.when(i >= 2)
            def _recycle():
                store_copy(i - 2, b).wait()

            slim(b)
            store_copy(i, b).start()

            @pl.when(i + 2 < b_per_w)
            def _next_gather():
                gather_copy(i + 2, b).start()

            return carry

        lax.fori_loop(0, b_per_w, body, 0)
        store_copy(b_per_w - 2, lax.rem(b_per_w - 2, 2)).wait()
        store_copy(b_per_w - 1, lax.rem(b_per_w - 1, 2)).wait()

    return gather_kernel


def kernel(input_ids, weight):
    B, L = input_ids.shape
    V, D = weight.shape
    info = plsc.get_sparse_core_info()
    ids32 = input_ids.astype(jnp.int32)
    return _make_gather(B, L, V, D, info.num_cores, info.num_subcores)(
        ids32, weight)


# speedup vs baseline: 1.7406x; 1.7406x over previous
import functools

import jax
import jax.numpy as jnp
from jax import lax
from jax.experimental import pallas as pl
from jax.experimental.pallas import tpu as pltpu
from jax.experimental.pallas import tpu_sc as plsc

_NBUF = 5
_DEPTH = 3


def _make_gather(B, L, V, D, num_cores, num_subcores):
    NW = num_cores * num_subcores
    b_per_w = B // NW
    n_per_w = b_per_w * L
    mesh = plsc.VectorSubcoreMesh(core_axis_name="c", subcore_axis_name="s")

    @functools.partial(
        pl.kernel,
        mesh=mesh,
        out_type=jax.ShapeDtypeStruct((B, L, D), jnp.float32),
        scratch_types=[
            pltpu.VMEM((n_per_w,), jnp.int32),
            pltpu.VMEM((_NBUF, L, D), jnp.float32),
            pltpu.SemaphoreType.DMA((_NBUF,)),
            pltpu.SemaphoreType.DMA((_NBUF,)),
        ],
        compiler_params=pltpu.CompilerParams(use_tc_tiling_on_sc=False),
    )
    def gather_kernel(idx_hbm, table_hbm, out_hbm, idx_v, rows_v, gsem, ssem):
        wid = lax.axis_index("s") * num_cores + lax.axis_index("c")
        base = wid * n_per_w
        brow0 = wid * b_per_w

        pltpu.sync_copy(idx_hbm.at[pl.ds(base, n_per_w)], idx_v)

        def gather_copy(i, b):
            return pltpu.make_async_copy(
                table_hbm.at[idx_v.at[pl.ds(i * L, L)]], rows_v.at[b], gsem.at[b])

        def store_copy(i, b):
            return pltpu.make_async_copy(
                rows_v.at[b], out_hbm.at[brow0 + i], ssem.at[b])

        for j in range(_DEPTH):
            gather_copy(j, j).start()

        def body(i, carry):
            b = lax.rem(i, _NBUF)
            gather_copy(i, b).wait()
            store_copy(i, b).start()

            @pl.when(i + _DEPTH < b_per_w)
            def _next_gather():
                b2 = lax.rem(i + _DEPTH, _NBUF)

                @pl.when(i >= _NBUF - _DEPTH)
                def _recycle():
                    store_copy(i - (_NBUF - _DEPTH), b2).wait()

                gather_copy(i + _DEPTH, b2).start()

            return carry

        lax.fori_loop(0, b_per_w, body, 0)
        for j in range(b_per_w - _NBUF, b_per_w):
            store_copy(j, j % _NBUF).wait()

    return gather_kernel


def kernel(input_ids, weight):
    B, L = input_ids.shape
    V, D = weight.shape
    info = plsc.get_sparse_core_info()
    flat_idx = input_ids.reshape(B * L).astype(jnp.int32)
    return _make_gather(B, L, V, D, info.num_cores, info.num_subcores)(
        flat_idx, weight)


# speedup vs baseline: 1.7451x; 1.0026x over previous
import functools

import jax
import jax.numpy as jnp
from jax import lax
from jax.experimental import pallas as pl
from jax.experimental.pallas import tpu as pltpu
from jax.experimental.pallas import tpu_sc as plsc

_NBUF = 5
_DEPTH = 3


def _make_gather(B, L, V, D, num_cores, num_subcores):
    NW = num_cores * num_subcores
    b_per_w = B // NW
    n_per_w = b_per_w * L
    mesh = plsc.VectorSubcoreMesh(core_axis_name="c", subcore_axis_name="s")

    @functools.partial(
        pl.kernel,
        mesh=mesh,
        out_type=jax.ShapeDtypeStruct((B, L, D), jnp.float32),
        scratch_types=[
            pltpu.VMEM((b_per_w, L), jnp.int32),
            pltpu.VMEM((_NBUF, L, D), jnp.float32),
            pltpu.SemaphoreType.DMA((_NBUF,)),
            pltpu.SemaphoreType.DMA((_NBUF,)),
        ],
        compiler_params=pltpu.CompilerParams(use_tc_tiling_on_sc=False),
    )
    def gather_kernel(idx_hbm, table_hbm, out_hbm, idx_v, rows_v, gsem, ssem):
        wid = lax.axis_index("s") * num_cores + lax.axis_index("c")
        brow0 = wid * b_per_w

        pltpu.sync_copy(idx_hbm.at[pl.ds(brow0, b_per_w), :], idx_v)

        def gather_copy(i, b):
            return pltpu.make_async_copy(
                table_hbm.at[idx_v.at[i]], rows_v.at[b], gsem.at[b])

        def store_copy(i, b):
            return pltpu.make_async_copy(
                rows_v.at[b], out_hbm.at[brow0 + i], ssem.at[b])

        for j in range(_DEPTH):
            gather_copy(j, j).start()

        def body(i, carry):
            b = lax.rem(i, _NBUF)
            gather_copy(i, b).wait()
            store_copy(i, b).start()

            @pl.when(i + _DEPTH < b_per_w)
            def _next_gather():
                b2 = lax.rem(i + _DEPTH, _NBUF)

                @pl.when(i >= _NBUF - _DEPTH)
                def _recycle():
                    store_copy(i - (_NBUF - _DEPTH), b2).wait()

                gather_copy(i + _DEPTH, b2).start()

            return carry

        lax.fori_loop(0, b_per_w, body, 0)
        for j in range(b_per_w - _NBUF, b_per_w):
            store_copy(j, j % _NBUF).wait()

    return gather_kernel


def kernel(input_ids, weight):
    B, L = input_ids.shape
    V, D = weight.shape
    info = plsc.get_sparse_core_info()
    return _make_gather(B, L, V, D, info.num_cores, info.num_subcores)(
        input_ids.astype(jnp.int32), weight)
